# baseline (speedup 1.0000x reference)
import math
import jax
import jax.numpy as jnp
from jax.experimental import pallas as pl

N = 5000
C = 91
F = 1024
IMG_H = 800.0
IMG_W = 800.0
SCORE_THRESH = 0.05
NMS_THRESH = 0.5
MAX_BOXES = 100
K_CAND = 1000
K_PAD = 1024
K_ROW = 24
BBOX_XFORM_CLIP = math.log(1000.0 / 16.0)


def _decode_kernel(logits_ref, dx_ref, dy_ref, dw_ref, dh_ref, prop_ref,
                   masked_ref, x1_ref, y1_ref, x2_ref, y2_ref,
                   vals_ref, cols_ref):
    logits = logits_ref[...]
    m = jnp.max(logits, axis=1, keepdims=True)
    e = jnp.exp(logits - m)
    s = jnp.sum(e, axis=1, keepdims=True)
    p = e / s

    px1 = prop_ref[:, 0:1]
    py1 = prop_ref[:, 1:2]
    px2 = prop_ref[:, 2:3]
    py2 = prop_ref[:, 3:4]
    widths = px2 - px1
    heights = py2 - py1
    ctr_x = px1 + 0.5 * widths
    ctr_y = py1 + 0.5 * heights

    dx = dx_ref[...] / 10.0
    dy = dy_ref[...] / 10.0
    dw = jnp.minimum(dw_ref[...] / 5.0, BBOX_XFORM_CLIP)
    dh = jnp.minimum(dh_ref[...] / 5.0, BBOX_XFORM_CLIP)

    pctr_x = dx * widths + ctr_x
    pctr_y = dy * heights + ctr_y
    pw = jnp.exp(dw) * widths
    ph = jnp.exp(dh) * heights

    x1 = jnp.clip(pctr_x - 0.5 * pw, 0.0, IMG_W)
    y1 = jnp.clip(pctr_y - 0.5 * ph, 0.0, IMG_H)
    x2 = jnp.clip(pctr_x + 0.5 * pw, 0.0, IMG_W)
    y2 = jnp.clip(pctr_y + 0.5 * ph, 0.0, IMG_H)

    ws = x2 - x1
    hs = y2 - y1
    valid = (p > SCORE_THRESH) & (ws >= 0.01) & (hs >= 0.01)
    masked = jnp.where(valid, p, -1.0)
    masked_ref[...] = masked
    x1_ref[...] = x1
    y1_ref[...] = y1
    x2_ref[...] = x2
    y2_ref[...] = y2

    work = masked[:, 1:]
    col_iota = jax.lax.broadcasted_iota(jnp.int32, (N, C - 1), 1)
    out_iota = jax.lax.broadcasted_iota(jnp.int32, (N, K_ROW), 1)

    def ext(k, carry):
        work, vals, cols = carry
        mv = jnp.max(work, axis=1, keepdims=True)
        mc = jnp.min(jnp.where(work == mv, col_iota, C - 1), axis=1,
                     keepdims=True)
        at_k = out_iota == k
        vals = jnp.where(at_k, mv, vals)
        cols = jnp.where(at_k, mc, cols)
        work = jnp.where(col_iota == mc, -2.0, work)
        return work, vals, cols

    _, vals, cols = jax.lax.fori_loop(
        0, K_ROW, ext,
        (work, jnp.full((N, K_ROW), -2.0, jnp.float32),
         jnp.zeros((N, K_ROW), jnp.int32)))
    vals_ref[...] = vals
    cols_ref[...] = cols


def _nms_kernel(x1_ref, y1_ref, x2_ref, y2_ref, scores_ref, labels_ref,
                top_scores_ref, top_pos_ref):
    gidx = (jax.lax.broadcasted_iota(jnp.int32, (8, 128), 0) * 128
            + jax.lax.broadcasted_iota(jnp.int32, (8, 128), 1))

    off = labels_ref[...] * (IMG_W + IMG_H + 1.0)
    x1 = x1_ref[...] + off
    y1 = y1_ref[...] + off
    x2 = x2_ref[...] + off
    y2 = y2_ref[...] + off
    scores = scores_ref[...]
    area = (x2 - x1) * (y2 - y1)

    def body(i, supp):
        at_i = gidx == i
        bx1 = jnp.max(jnp.where(at_i, x1, -3.4e38))
        by1 = jnp.max(jnp.where(at_i, y1, -3.4e38))
        bx2 = jnp.max(jnp.where(at_i, x2, -3.4e38))
        by2 = jnp.max(jnp.where(at_i, y2, -3.4e38))
        supp_i = jnp.max(jnp.where(at_i, supp, 0.0))
        bar = (bx2 - bx1) * (by2 - by1)

        iw = jnp.maximum(jnp.minimum(bx2, x2) - jnp.maximum(bx1, x1), 0.0)
        ih = jnp.maximum(jnp.minimum(by2, y2) - jnp.maximum(by1, y1), 0.0)
        inter = iw * ih
        union = jnp.maximum(bar + area - inter, 1e-9)
        iou = inter / union

        row = (iou > NMS_THRESH) & (gidx > i) & (supp_i < 0.5)
        return jnp.maximum(supp, jnp.where(row, 1.0, 0.0))

    supp = jax.lax.fori_loop(0, K_CAND, body, jnp.zeros((8, 128), jnp.float32))

    keep = (supp < 0.5) & (scores > SCORE_THRESH)
    masked = jnp.where(keep, scores, -1.0)

    out_lane = jax.lax.iota(jnp.int32, 128).reshape(1, 128)

    def sel_body(k, carry):
        masked_k, sc_row, pos_row = carry
        m = jnp.max(masked_k)
        pos = jnp.min(jnp.where(masked_k == m, gidx, K_PAD))
        at_k = out_lane == k
        sc_row = jnp.where(at_k, m, sc_row)
        pos_row = jnp.where(at_k, pos, pos_row)
        masked_k = jnp.where(gidx == pos, -3.0, masked_k)
        return masked_k, sc_row, pos_row

    _, sc_row, pos_row = jax.lax.fori_loop(
        0, MAX_BOXES, sel_body,
        (masked, jnp.full((1, 128), -4.0, jnp.float32),
         jnp.zeros((1, 128), jnp.int32)))

    top_scores_ref[...] = sc_row
    top_pos_ref[...] = pos_row


def kernel(class_logits, box_features, box_regression, proposals):
    rel = box_regression.reshape(N, C, 4)
    dx = rel[..., 0]
    dy = rel[..., 1]
    dw = rel[..., 2]
    dh = rel[..., 3]

    plane = jax.ShapeDtypeStruct((N, C), jnp.float32)
    masked_p, x1, y1, x2, y2, vals, cols = pl.pallas_call(
        _decode_kernel,
        out_shape=[plane] * 5 + [jax.ShapeDtypeStruct((N, K_ROW), jnp.float32),
                                 jax.ShapeDtypeStruct((N, K_ROW), jnp.int32)],
    )(class_logits, dx, dy, dw, dh, proposals)

    flat24 = vals.reshape(-1)
    cs, pos = jax.lax.top_k(flat24, K_CAND)
    row24 = pos // K_ROW
    col90 = cols.reshape(-1)[pos]
    cand_idx_fast = row24 * (C - 1) + col90

    need_full = cs[K_CAND - 1] < 0.0

    def full_path(_):
        fs, fidx = jax.lax.top_k(masked_p[:, 1:].reshape(-1), K_CAND)
        return fs, fidx

    def fast_path(_):
        return cs, cand_idx_fast

    cand_scores, cand_idx = jax.lax.cond(need_full, full_path, fast_path,
                                         operand=None)

    row = cand_idx // (C - 1)
    col = (cand_idx % (C - 1)) + 1
    cx1 = x1[row, col]
    cy1 = y1[row, col]
    cx2 = x2[row, col]
    cy2 = y2[row, col]
    cand_labels = col

    def pad_tile(v, fill):
        return jnp.pad(v, (0, K_PAD - K_CAND),
                       constant_values=fill).reshape(8, 128)

    top_scores_row, top_pos_row = pl.pallas_call(
        _nms_kernel,
        out_shape=[jax.ShapeDtypeStruct((1, 128), jnp.float32),
                   jax.ShapeDtypeStruct((1, 128), jnp.int32)],
    )(pad_tile(cx1, 0.0), pad_tile(cy1, 0.0),
      pad_tile(cx2, 0.0), pad_tile(cy2, 0.0),
      pad_tile(cand_scores, -2.0),
      pad_tile(cand_labels.astype(jnp.float32), 0.0))

    top_scores = top_scores_row[0, :MAX_BOXES]
    top_pos = top_pos_row[0, :MAX_BOXES]

    fi = cand_idx[top_pos]
    boxes_out = jnp.stack([cx1[top_pos], cy1[top_pos],
                           cx2[top_pos], cy2[top_pos]], axis=-1)
    feats_out = box_features[fi // (C - 1)]
    labels_out = (fi % (C - 1)) + 1
    return feats_out, boxes_out, top_scores, labels_out


# speedup vs baseline: 1.0302x; 1.0302x over previous
import math
import jax
import jax.numpy as jnp
from jax.experimental import pallas as pl
from jax.experimental.pallas import tpu as pltpu

N = 5000
C = 91
F = 1024
IMG_H = 800.0
IMG_W = 800.0
SCORE_THRESH = 0.05
NMS_THRESH = 0.5
MAX_BOXES = 100
K_CAND = 1000
K_PAD = 1024
K_ROW = 24
BBOX_XFORM_CLIP = math.log(1000.0 / 16.0)


def _decode_kernel(logits_ref, dx_ref, dy_ref, dw_ref, dh_ref, prop_ref,
                   masked_ref, x1_ref, y1_ref, x2_ref, y2_ref,
                   vals_ref, cols_ref):
    logits = logits_ref[...]
    m = jnp.max(logits, axis=1, keepdims=True)
    e = jnp.exp(logits - m)
    s = jnp.sum(e, axis=1, keepdims=True)
    p = e / s

    px1 = prop_ref[:, 0:1]
    py1 = prop_ref[:, 1:2]
    px2 = prop_ref[:, 2:3]
    py2 = prop_ref[:, 3:4]
    widths = px2 - px1
    heights = py2 - py1
    ctr_x = px1 + 0.5 * widths
    ctr_y = py1 + 0.5 * heights

    dx = dx_ref[...] / 10.0
    dy = dy_ref[...] / 10.0
    dw = jnp.minimum(dw_ref[...] / 5.0, BBOX_XFORM_CLIP)
    dh = jnp.minimum(dh_ref[...] / 5.0, BBOX_XFORM_CLIP)

    pctr_x = dx * widths + ctr_x
    pctr_y = dy * heights + ctr_y
    pw = jnp.exp(dw) * widths
    ph = jnp.exp(dh) * heights

    x1 = jnp.clip(pctr_x - 0.5 * pw, 0.0, IMG_W)
    y1 = jnp.clip(pctr_y - 0.5 * ph, 0.0, IMG_H)
    x2 = jnp.clip(pctr_x + 0.5 * pw, 0.0, IMG_W)
    y2 = jnp.clip(pctr_y + 0.5 * ph, 0.0, IMG_H)

    ws = x2 - x1
    hs = y2 - y1
    valid = (p > SCORE_THRESH) & (ws >= 0.01) & (hs >= 0.01)
    masked = jnp.where(valid, p, -1.0)
    masked_ref[...] = masked
    x1_ref[...] = x1
    y1_ref[...] = y1
    x2_ref[...] = x2
    y2_ref[...] = y2

    work = masked[:, 1:]
    col_iota = jax.lax.broadcasted_iota(jnp.int32, (N, C - 1), 1)
    out_iota = jax.lax.broadcasted_iota(jnp.int32, (N, K_ROW), 1)

    def ext(k, carry):
        work, vals, cols = carry
        mv = jnp.max(work, axis=1, keepdims=True)
        mc = jnp.min(jnp.where(work == mv, col_iota, C - 1), axis=1,
                     keepdims=True)
        at_k = out_iota == k
        vals = jnp.where(at_k, mv, vals)
        cols = jnp.where(at_k, mc, cols)
        work = jnp.where(col_iota == mc, -2.0, work)
        return work, vals, cols

    _, vals, cols = jax.lax.fori_loop(
        0, K_ROW, ext,
        (work, jnp.full((N, K_ROW), -2.0, jnp.float32),
         jnp.zeros((N, K_ROW), jnp.int32)))
    vals_ref[...] = vals
    cols_ref[...] = cols


def _nms_kernel(x1s_ref, y1s_ref, x2s_ref, y2s_ref, labs_ref,
                x1_ref, y1_ref, x2_ref, y2_ref, scores_ref, labels_ref,
                top_scores_ref, top_pos_ref):
    gidx = (jax.lax.broadcasted_iota(jnp.int32, (8, 128), 0) * 128
            + jax.lax.broadcasted_iota(jnp.int32, (8, 128), 1))

    off = labels_ref[...] * (IMG_W + IMG_H + 1.0)
    x1 = x1_ref[...] + off
    y1 = y1_ref[...] + off
    x2 = x2_ref[...] + off
    y2 = y2_ref[...] + off
    scores = scores_ref[...]
    area = (x2 - x1) * (y2 - y1)

    def body(i, supp):
        at_i = gidx == i
        off_i = labs_ref[i] * (IMG_W + IMG_H + 1.0)
        bx1 = x1s_ref[i] + off_i
        by1 = y1s_ref[i] + off_i
        bx2 = x2s_ref[i] + off_i
        by2 = y2s_ref[i] + off_i
        supp_i = jnp.max(jnp.where(at_i, supp, 0.0))
        bar = (bx2 - bx1) * (by2 - by1)

        iw = jnp.maximum(jnp.minimum(bx2, x2) - jnp.maximum(bx1, x1), 0.0)
        ih = jnp.maximum(jnp.minimum(by2, y2) - jnp.maximum(by1, y1), 0.0)
        inter = iw * ih
        union = jnp.maximum(bar + area - inter, 1e-9)
        iou = inter / union

        row = (iou > NMS_THRESH) & (gidx > i) & (supp_i < 0.5)
        return jnp.maximum(supp, jnp.where(row, 1.0, 0.0))

    supp = jax.lax.fori_loop(0, K_CAND, body, jnp.zeros((8, 128), jnp.float32))

    keep = (supp < 0.5) & (scores > SCORE_THRESH)
    masked = jnp.where(keep, scores, -1.0)

    out_lane = jax.lax.iota(jnp.int32, 128).reshape(1, 128)

    def sel_body(k, carry):
        masked_k, sc_row, pos_row = carry
        m = jnp.max(masked_k)
        pos = jnp.min(jnp.where(masked_k == m, gidx, K_PAD))
        at_k = out_lane == k
        sc_row = jnp.where(at_k, m, sc_row)
        pos_row = jnp.where(at_k, pos, pos_row)
        masked_k = jnp.where(gidx == pos, -3.0, masked_k)
        return masked_k, sc_row, pos_row

    _, sc_row, pos_row = jax.lax.fori_loop(
        0, MAX_BOXES, sel_body,
        (masked, jnp.full((1, 128), -4.0, jnp.float32),
         jnp.zeros((1, 128), jnp.int32)))

    top_scores_ref[...] = sc_row
    top_pos_ref[...] = pos_row


def kernel(class_logits, box_features, box_regression, proposals):
    rel = box_regression.reshape(N, C, 4)
    dx = rel[..., 0]
    dy = rel[..., 1]
    dw = rel[..., 2]
    dh = rel[..., 3]

    plane = jax.ShapeDtypeStruct((N, C), jnp.float32)
    masked_p, x1, y1, x2, y2, vals, cols = pl.pallas_call(
        _decode_kernel,
        out_shape=[plane] * 5 + [jax.ShapeDtypeStruct((N, K_ROW), jnp.float32),
                                 jax.ShapeDtypeStruct((N, K_ROW), jnp.int32)],
    )(class_logits, dx, dy, dw, dh, proposals)

    flat24 = vals.reshape(-1)
    cs, pos = jax.lax.top_k(flat24, K_CAND)
    row24 = pos // K_ROW
    col90 = cols.reshape(-1)[pos]
    cand_idx_fast = row24 * (C - 1) + col90

    need_full = cs[K_CAND - 1] < 0.0

    def full_path(_):
        fs, fidx = jax.lax.top_k(masked_p[:, 1:].reshape(-1), K_CAND)
        return fs, fidx

    def fast_path(_):
        return cs, cand_idx_fast

    cand_scores, cand_idx = jax.lax.cond(need_full, full_path, fast_path,
                                         operand=None)

    row = cand_idx // (C - 1)
    col = (cand_idx % (C - 1)) + 1
    cx1 = x1[row, col]
    cy1 = y1[row, col]
    cx2 = x2[row, col]
    cy2 = y2[row, col]
    cand_labels = col

    def pad_tile(v, fill):
        return jnp.pad(v, (0, K_PAD - K_CAND),
                       constant_values=fill).reshape(8, 128)

    def pad_flat(v, fill):
        return jnp.pad(v, (0, K_PAD - K_CAND), constant_values=fill)

    labf = cand_labels.astype(jnp.float32)
    smem_spec = pl.BlockSpec(memory_space=pltpu.SMEM)
    vmem_spec = pl.BlockSpec(memory_space=pltpu.VMEM)
    top_scores_row, top_pos_row = pl.pallas_call(
        _nms_kernel,
        in_specs=[smem_spec] * 5 + [vmem_spec] * 6,
        out_shape=[jax.ShapeDtypeStruct((1, 128), jnp.float32),
                   jax.ShapeDtypeStruct((1, 128), jnp.int32)],
    )(pad_flat(cx1, 0.0), pad_flat(cy1, 0.0),
      pad_flat(cx2, 0.0), pad_flat(cy2, 0.0), pad_flat(labf, 0.0),
      pad_tile(cx1, 0.0), pad_tile(cy1, 0.0),
      pad_tile(cx2, 0.0), pad_tile(cy2, 0.0),
      pad_tile(cand_scores, -2.0),
      pad_tile(labf, 0.0))

    top_scores = top_scores_row[0, :MAX_BOXES]
    top_pos = top_pos_row[0, :MAX_BOXES]

    fi = cand_idx[top_pos]
    boxes_out = jnp.stack([cx1[top_pos], cy1[top_pos],
                           cx2[top_pos], cy2[top_pos]], axis=-1)
    feats_out = box_features[fi // (C - 1)]
    labels_out = (fi % (C - 1)) + 1
    return feats_out, boxes_out, top_scores, labels_out


# speedup vs baseline: 1.3354x; 1.2962x over previous
import math
import jax
import jax.numpy as jnp
from jax.experimental import pallas as pl
from jax.experimental.pallas import tpu as pltpu

N = 5000
C = 91
F = 1024
IMG_H = 800.0
IMG_W = 800.0
SCORE_THRESH = 0.05
NMS_THRESH = 0.5
MAX_BOXES = 100
K_CAND = 1000
K_PAD = 1024
K_ROW = 24
BBOX_XFORM_CLIP = math.log(1000.0 / 16.0)


def _decode_kernel(logits_ref, dx_ref, dy_ref, dw_ref, dh_ref, prop_ref,
                   masked_ref, x1_ref, y1_ref, x2_ref, y2_ref,
                   vals_ref, cols_ref):
    logits = logits_ref[...]
    m = jnp.max(logits, axis=1, keepdims=True)
    e = jnp.exp(logits - m)
    s = jnp.sum(e, axis=1, keepdims=True)
    p = e / s

    px1 = prop_ref[:, 0:1]
    py1 = prop_ref[:, 1:2]
    px2 = prop_ref[:, 2:3]
    py2 = prop_ref[:, 3:4]
    widths = px2 - px1
    heights = py2 - py1
    ctr_x = px1 + 0.5 * widths
    ctr_y = py1 + 0.5 * heights

    dx = dx_ref[...] / 10.0
    dy = dy_ref[...] / 10.0
    dw = jnp.minimum(dw_ref[...] / 5.0, BBOX_XFORM_CLIP)
    dh = jnp.minimum(dh_ref[...] / 5.0, BBOX_XFORM_CLIP)

    pctr_x = dx * widths + ctr_x
    pctr_y = dy * heights + ctr_y
    pw = jnp.exp(dw) * widths
    ph = jnp.exp(dh) * heights

    x1 = jnp.clip(pctr_x - 0.5 * pw, 0.0, IMG_W)
    y1 = jnp.clip(pctr_y - 0.5 * ph, 0.0, IMG_H)
    x2 = jnp.clip(pctr_x + 0.5 * pw, 0.0, IMG_W)
    y2 = jnp.clip(pctr_y + 0.5 * ph, 0.0, IMG_H)

    ws = x2 - x1
    hs = y2 - y1
    valid = (p > SCORE_THRESH) & (ws >= 0.01) & (hs >= 0.01)
    masked = jnp.where(valid, p, -1.0)
    masked_ref[...] = masked
    x1_ref[...] = x1
    y1_ref[...] = y1
    x2_ref[...] = x2
    y2_ref[...] = y2

    work = masked[:, 1:]
    col_iota = jax.lax.broadcasted_iota(jnp.int32, (N, C - 1), 1)
    out_iota = jax.lax.broadcasted_iota(jnp.int32, (N, K_ROW), 1)

    def ext(k, carry):
        work, vals, cols = carry
        mv = jnp.max(work, axis=1, keepdims=True)
        mc = jnp.min(jnp.where(work == mv, col_iota, C - 1), axis=1,
                     keepdims=True)
        at_k = out_iota == k
        vals = jnp.where(at_k, mv, vals)
        cols = jnp.where(at_k, mc, cols)
        work = jnp.where(col_iota == mc, -2.0, work)
        return work, vals, cols

    _, vals, cols = jax.lax.fori_loop(
        0, K_ROW, ext,
        (work, jnp.full((N, K_ROW), -2.0, jnp.float32),
         jnp.zeros((N, K_ROW), jnp.int32)))
    vals_ref[...] = vals
    cols_ref[...] = cols


N_SEG = 128
SEG_L = 960
K_SEG = 32


def _seg_topk_kernel(vals_ref, seg_vals_ref, seg_pos_ref):
    work = vals_ref[...]
    lane = jax.lax.broadcasted_iota(jnp.int32, (N_SEG, SEG_L), 1)
    out_iota = jax.lax.broadcasted_iota(jnp.int32, (N_SEG, K_SEG), 1)

    def ext(k, carry):
        work, sv, sp = carry
        mv = jnp.max(work, axis=1, keepdims=True)
        mc = jnp.min(jnp.where(work == mv, lane, SEG_L), axis=1,
                     keepdims=True)
        at_k = out_iota == k
        sv = jnp.where(at_k, mv, sv)
        sp = jnp.where(at_k, mc, sp)
        work = jnp.where(lane == mc, -5.0, work)
        return work, sv, sp

    _, sv, sp = jax.lax.fori_loop(
        0, K_SEG, ext,
        (work, jnp.full((N_SEG, K_SEG), -5.0, jnp.float32),
         jnp.zeros((N_SEG, K_SEG), jnp.int32)))
    seg_vals_ref[...] = sv
    seg_pos_ref[...] = sp


def _nms_kernel(x1s_ref, y1s_ref, x2s_ref, y2s_ref, labs_ref,
                x1_ref, y1_ref, x2_ref, y2_ref, scores_ref, labels_ref,
                top_scores_ref, top_pos_ref):
    gidx = (jax.lax.broadcasted_iota(jnp.int32, (8, 128), 0) * 128
            + jax.lax.broadcasted_iota(jnp.int32, (8, 128), 1))

    off = labels_ref[...] * (IMG_W + IMG_H + 1.0)
    x1 = x1_ref[...] + off
    y1 = y1_ref[...] + off
    x2 = x2_ref[...] + off
    y2 = y2_ref[...] + off
    scores = scores_ref[...]
    area = (x2 - x1) * (y2 - y1)

    def body(i, supp):
        at_i = gidx == i
        off_i = labs_ref[i] * (IMG_W + IMG_H + 1.0)
        bx1 = x1s_ref[i] + off_i
        by1 = y1s_ref[i] + off_i
        bx2 = x2s_ref[i] + off_i
        by2 = y2s_ref[i] + off_i
        supp_i = jnp.max(jnp.where(at_i, supp, 0.0))
        bar = (bx2 - bx1) * (by2 - by1)

        iw = jnp.maximum(jnp.minimum(bx2, x2) - jnp.maximum(bx1, x1), 0.0)
        ih = jnp.maximum(jnp.minimum(by2, y2) - jnp.maximum(by1, y1), 0.0)
        inter = iw * ih
        union = jnp.maximum(bar + area - inter, 1e-9)
        t_eff = jnp.where(supp_i < 0.5, NMS_THRESH, 3.4e38)

        row = (inter > t_eff * union) & (gidx > i)
        return jnp.maximum(supp, jnp.where(row, 1.0, 0.0))

    supp = jax.lax.fori_loop(0, K_CAND, body, jnp.zeros((8, 128), jnp.float32))

    keep = (supp < 0.5) & (scores > SCORE_THRESH)
    masked = jnp.where(keep, scores, -1.0)

    out_lane = jax.lax.iota(jnp.int32, 128).reshape(1, 128)

    def sel_body(k, carry):
        masked_k, sc_row, pos_row = carry
        m = jnp.max(masked_k)
        pos = jnp.min(jnp.where(masked_k == m, gidx, K_PAD))
        at_k = out_lane == k
        sc_row = jnp.where(at_k, m, sc_row)
        pos_row = jnp.where(at_k, pos, pos_row)
        masked_k = jnp.where(gidx == pos, -3.0, masked_k)
        return masked_k, sc_row, pos_row

    _, sc_row, pos_row = jax.lax.fori_loop(
        0, MAX_BOXES, sel_body,
        (masked, jnp.full((1, 128), -4.0, jnp.float32),
         jnp.zeros((1, 128), jnp.int32)))

    top_scores_ref[...] = sc_row
    top_pos_ref[...] = pos_row


def kernel(class_logits, box_features, box_regression, proposals):
    rel = box_regression.reshape(N, C, 4)
    dx = rel[..., 0]
    dy = rel[..., 1]
    dw = rel[..., 2]
    dh = rel[..., 3]

    plane = jax.ShapeDtypeStruct((N, C), jnp.float32)
    masked_p, x1, y1, x2, y2, vals, cols = pl.pallas_call(
        _decode_kernel,
        out_shape=[plane] * 5 + [jax.ShapeDtypeStruct((N, K_ROW), jnp.float32),
                                 jax.ShapeDtypeStruct((N, K_ROW), jnp.int32)],
    )(class_logits, dx, dy, dw, dh, proposals)

    flat24 = vals.reshape(-1)
    padded = jnp.pad(flat24, (0, N_SEG * SEG_L - N * K_ROW),
                     constant_values=-2.0).reshape(N_SEG, SEG_L)
    seg_vals, seg_pos = pl.pallas_call(
        _seg_topk_kernel,
        out_shape=[jax.ShapeDtypeStruct((N_SEG, K_SEG), jnp.float32),
                   jax.ShapeDtypeStruct((N_SEG, K_SEG), jnp.int32)],
    )(padded)

    u = seg_vals.reshape(-1)
    cs, upos = jax.lax.top_k(u, K_CAND)
    gpos = (upos // K_SEG) * SEG_L + seg_pos.reshape(-1)[upos]
    row24 = gpos // K_ROW
    col90 = cols.reshape(-1)[gpos]
    cand_idx_fast = row24 * (C - 1) + col90
    t = cs[K_CAND - 1]

    need_full = t < 0.0
    need_mid = jnp.any(seg_vals[:, K_SEG - 1] >= t)

    def full_path(_):
        fs, fidx = jax.lax.top_k(masked_p[:, 1:].reshape(-1), K_CAND)
        return fs, fidx

    def mid_path(_):
        ms, mpos = jax.lax.top_k(flat24, K_CAND)
        return ms, (mpos // K_ROW) * (C - 1) + cols.reshape(-1)[mpos]

    def fast_path(_):
        return cs, cand_idx_fast

    cand_scores, cand_idx = jax.lax.cond(
        need_full, full_path,
        lambda _: jax.lax.cond(need_mid, mid_path, fast_path, operand=None),
        operand=None)

    row = cand_idx // (C - 1)
    col = (cand_idx % (C - 1)) + 1
    cx1 = x1[row, col]
    cy1 = y1[row, col]
    cx2 = x2[row, col]
    cy2 = y2[row, col]
    cand_labels = col

    def pad_tile(v, fill):
        return jnp.pad(v, (0, K_PAD - K_CAND),
                       constant_values=fill).reshape(8, 128)

    def pad_flat(v, fill):
        return jnp.pad(v, (0, K_PAD - K_CAND), constant_values=fill)

    labf = cand_labels.astype(jnp.float32)
    smem_spec = pl.BlockSpec(memory_space=pltpu.SMEM)
    vmem_spec = pl.BlockSpec(memory_space=pltpu.VMEM)
    top_scores_row, top_pos_row = pl.pallas_call(
        _nms_kernel,
        in_specs=[smem_spec] * 5 + [vmem_spec] * 6,
        out_shape=[jax.ShapeDtypeStruct((1, 128), jnp.float32),
                   jax.ShapeDtypeStruct((1, 128), jnp.int32)],
    )(pad_flat(cx1, 0.0), pad_flat(cy1, 0.0),
      pad_flat(cx2, 0.0), pad_flat(cy2, 0.0), pad_flat(labf, 0.0),
      pad_tile(cx1, 0.0), pad_tile(cy1, 0.0),
      pad_tile(cx2, 0.0), pad_tile(cy2, 0.0),
      pad_tile(cand_scores, -2.0),
      pad_tile(labf, 0.0))

    top_scores = top_scores_row[0, :MAX_BOXES]
    top_pos = top_pos_row[0, :MAX_BOXES]

    fi = cand_idx[top_pos]
    boxes_out = jnp.stack([cx1[top_pos], cy1[top_pos],
                           cx2[top_pos], cy2[top_pos]], axis=-1)
    feats_out = box_features[fi // (C - 1)]
    labels_out = (fi % (C - 1)) + 1
    return feats_out, boxes_out, top_scores, labels_out
